# baseline (speedup 1.0000x reference)
import jax
import jax.numpy as jnp
from jax import lax
from jax.experimental import pallas as pl
from jax.experimental.pallas import tpu as pltpu
from jax.experimental.pallas import tpu_sc as plsc

H = 768
NW = 32
SEG = 1024
CAP = 128
L = 16
BLK = 1 * SEG
SPB = BLK // SEG


_SC_MESH = plsc.VectorSubcoreMesh(
    core_axis_name="c", subcore_axis_name="s", num_cores=2, num_subcores=16
)


def _sc_compact(mask_hbm, idx_hbm, cnt_hbm, mask_v, idxl_v, cnt_v):
    wid = lax.axis_index("s") * 2 + lax.axis_index("c")
    base = wid * SEG
    pltpu.sync_copy(mask_hbm.at[pl.ds(base, SEG)], mask_v)

    zeros = jnp.zeros((L,), jnp.int32)
    for i in range(CAP // L):
        idxl_v[pl.ds(i * L, L)] = zeros

    lane = lax.iota(jnp.int32, L)

    def body(c, off_vec):
        mchunk = mask_v[pl.ds(c * L, L)]
        sel = mchunk == 1
        seli = jnp.where(sel, 1, 0).astype(jnp.int32)
        pos = off_vec + plsc.cumsum(seli) - 1
        okay = sel & (pos < CAP)
        posc = jnp.minimum(pos, CAP - 1)
        localpos = c * L + lane
        plsc.store_scatter(idxl_v, [posc], localpos, mask=okay)
        return off_vec + plsc.all_reduce_population_count(sel)

    n_vec = lax.fori_loop(0, SEG // L, body, jnp.zeros((L,), jnp.int32),
                          unroll=4)

    pltpu.sync_copy(idxl_v, idx_hbm.at[wid, 0])
    cnt_v[...] = n_vec
    pltpu.sync_copy(cnt_v, cnt_hbm.at[wid])


def _ln(z, g, be):
    mean = jnp.mean(z, axis=-1, keepdims=True)
    zc = z - mean
    var = jnp.mean(zc * zc, axis=-1, keepdims=True)
    return zc * lax.rsqrt(var + 1e-5) * g + be


def _bf16_dot(a, bmat):
    return lax.dot_general(
        a, bmat, (((1,), (0,)), ((), ())), preferred_element_type=jnp.float32
    )


def _sparse_body(cnt_smem, x_ref, idx_ref, wt_ref, b_ref, g_ref,
                 be_ref, o_ref):
    i = pl.program_id(0)
    rows = lax.broadcasted_iota(jnp.int32, (SEG, CAP), 0)
    kio = lax.broadcasted_iota(jnp.int32, (SEG, CAP), 1)
    cols = lax.broadcasted_iota(jnp.int32, (CAP, SEG), 1)
    krow = lax.broadcasted_iota(jnp.int32, (CAP, SEG), 0)
    xgs = []
    ps = []
    for s in range(SPB):
        n = cnt_smem[SPB * i + s, 0]
        idxr = idx_ref[s]
        idxc = idxr.reshape(CAP, 1)
        gmat = ((cols == idxc) & (krow < n)).astype(jnp.bfloat16)
        xb = x_ref[s * SEG:(s + 1) * SEG, :].astype(jnp.bfloat16)
        xgs.append(_bf16_dot(gmat, xb))
        ps.append(((rows == idxr) & (kio < n)).astype(jnp.bfloat16))
    xg = jnp.concatenate(xgs, axis=0)
    soft = _bf16_dot(xg.astype(jnp.bfloat16), wt_ref[...]) + b_ref[...]
    normed = _ln(xg + soft, g_ref[...], be_ref[...])
    d = (normed - xg).astype(jnp.bfloat16)
    for s in range(SPB):
        o_ref[s * SEG:(s + 1) * SEG, :] = (
            x_ref[s * SEG:(s + 1) * SEG, :]
            + _bf16_dot(ps[s], d[s * CAP:(s + 1) * CAP])
        )


def _dense_body(x_ref, m_ref, wt_ref, b_ref, g_ref, be_ref, o_ref):
    x = x_ref[...]
    soft = _bf16_dot(x.astype(jnp.bfloat16), wt_ref[...]) + b_ref[...]
    normed = _ln(x + soft, g_ref[...], be_ref[...])
    o_ref[...] = jnp.where(m_ref[...] == 1, normed, x)


def kernel(batch_embeddings, position_mask, W, b, gamma, beta):
    B, S, Hh = batch_embeddings.shape
    N = B * S
    x = batch_embeddings.reshape(N, Hh)
    mflat = position_mask.reshape(N).astype(jnp.int32)
    wt = W.T.astype(jnp.bfloat16)
    b2 = b.reshape(1, Hh)
    g2 = gamma.reshape(1, Hh)
    be2 = beta.reshape(1, Hh)

    sc = pl.kernel(
        _sc_compact,
        out_type=(
            jax.ShapeDtypeStruct((NW, 1, CAP), jnp.int32),
            jax.ShapeDtypeStruct((NW, L), jnp.int32),
        ),
        mesh=_SC_MESH,
        compiler_params=pltpu.CompilerParams(needs_layout_passes=False),
        scratch_types=[
            pltpu.VMEM((SEG,), jnp.int32),
            pltpu.VMEM((CAP,), jnp.int32),
            pltpu.VMEM((L,), jnp.int32),
        ],
    )
    idx, cnt = sc(mflat)

    def sparse_path(ops):
        xx, cc, ii = ops
        return pl.pallas_call(
            _sparse_body,
            grid=(N // BLK,),
            in_specs=[
                pl.BlockSpec(memory_space=pltpu.SMEM),
                pl.BlockSpec((BLK, Hh), lambda i: (i, 0)),
                pl.BlockSpec((SPB, 1, CAP), lambda i: (i, 0, 0)),
                pl.BlockSpec((Hh, Hh), lambda i: (0, 0)),
                pl.BlockSpec((1, Hh), lambda i: (0, 0)),
                pl.BlockSpec((1, Hh), lambda i: (0, 0)),
                pl.BlockSpec((1, Hh), lambda i: (0, 0)),
            ],
            out_specs=pl.BlockSpec((BLK, Hh), lambda i: (i, 0)),
            out_shape=jax.ShapeDtypeStruct((N, Hh), jnp.float32),
            compiler_params=pltpu.CompilerParams(
                dimension_semantics=("parallel",),
            ),
        )(cc, xx, ii, wt, b2, g2, be2)

    def dense_path(ops):
        xx, cc, ii = ops
        return pl.pallas_call(
            _dense_body,
            grid=(N // BLK,),
            in_specs=[
                pl.BlockSpec((BLK, Hh), lambda i: (i, 0)),
                pl.BlockSpec((BLK, 1), lambda i: (i, 0)),
                pl.BlockSpec((Hh, Hh), lambda i: (0, 0)),
                pl.BlockSpec((1, Hh), lambda i: (0, 0)),
                pl.BlockSpec((1, Hh), lambda i: (0, 0)),
                pl.BlockSpec((1, Hh), lambda i: (0, 0)),
            ],
            out_specs=pl.BlockSpec((BLK, Hh), lambda i: (i, 0)),
            out_shape=jax.ShapeDtypeStruct((N, Hh), jnp.float32),
            compiler_params=pltpu.CompilerParams(
                dimension_semantics=("arbitrary",),
            ),
        )(xx, mflat.reshape(N, 1), wt, b2, g2, be2)

    overflow = jnp.any(cnt[:, 0] > CAP)
    out = lax.cond(overflow, dense_path, sparse_path, (x, cnt, idx))
    return out.reshape(B, S, Hh)


# speedup vs baseline: 1.1311x; 1.1311x over previous
import jax
import jax.numpy as jnp
from jax import lax
from jax.experimental import pallas as pl
from jax.experimental.pallas import tpu as pltpu
from jax.experimental.pallas import tpu_sc as plsc

H = 768
NW = 32
SEG = 1024
CAP = 128
L = 16
BLK = 2 * SEG
SPB = BLK // SEG


_SC_MESH = plsc.VectorSubcoreMesh(
    core_axis_name="c", subcore_axis_name="s", num_cores=2, num_subcores=16
)


def _sc_compact(mask_hbm, idx_hbm, cnt_hbm, mask_v, idxl_v, cnt_v):
    wid = lax.axis_index("s") * 2 + lax.axis_index("c")
    base = wid * SEG
    pltpu.sync_copy(mask_hbm.at[pl.ds(base, SEG)], mask_v)

    zeros = jnp.zeros((L,), jnp.int32)
    for i in range(CAP // L):
        idxl_v[pl.ds(i * L, L)] = zeros

    lane = lax.iota(jnp.int32, L)

    def body(c, off_vec):
        mchunk = mask_v[pl.ds(c * L, L)]
        sel = mchunk == 1
        seli = jnp.where(sel, 1, 0).astype(jnp.int32)
        pos = off_vec + plsc.cumsum(seli) - 1
        okay = sel & (pos < CAP)
        posc = jnp.minimum(pos, CAP - 1)
        localpos = c * L + lane
        plsc.store_scatter(idxl_v, [posc], localpos, mask=okay)
        return off_vec + plsc.all_reduce_population_count(sel)

    n_vec = lax.fori_loop(0, SEG // L, body, jnp.zeros((L,), jnp.int32),
                          unroll=4)

    pltpu.sync_copy(idxl_v, idx_hbm.at[wid, 0])
    cnt_v[...] = n_vec
    pltpu.sync_copy(cnt_v, cnt_hbm.at[wid])


def _ln(z, g, be):
    mean = jnp.mean(z, axis=-1, keepdims=True)
    zc = z - mean
    var = jnp.mean(zc * zc, axis=-1, keepdims=True)
    return zc * lax.rsqrt(var + 1e-5) * g + be


def _bf16_dot(a, bmat):
    return lax.dot_general(
        a, bmat, (((1,), (0,)), ((), ())), preferred_element_type=jnp.float32
    )


def _sparse_body(cnt_smem, x_ref, idx_ref, wt_ref, b_ref, g_ref,
                 be_ref, o_ref):
    i = pl.program_id(0)
    rows = lax.broadcasted_iota(jnp.int32, (SEG, CAP), 0)
    kio = lax.broadcasted_iota(jnp.int32, (SEG, CAP), 1)
    cols = lax.broadcasted_iota(jnp.int32, (CAP, SEG), 1)
    krow = lax.broadcasted_iota(jnp.int32, (CAP, SEG), 0)
    xgs = []
    ps = []
    for s in range(SPB):
        n = cnt_smem[SPB * i + s, 0]
        idxr = idx_ref[s]
        idxc = idxr.reshape(CAP, 1)
        gmat = ((cols == idxc) & (krow < n)).astype(jnp.bfloat16)
        xb = x_ref[s * SEG:(s + 1) * SEG, :].astype(jnp.bfloat16)
        xgs.append(_bf16_dot(gmat, xb))
        ps.append(((rows == idxr) & (kio < n)).astype(jnp.bfloat16))
    xg = jnp.concatenate(xgs, axis=0)
    soft = _bf16_dot(xg.astype(jnp.bfloat16), wt_ref[...]) + b_ref[...]
    normed = _ln(xg + soft, g_ref[...], be_ref[...])
    d = (normed - xg).astype(jnp.bfloat16)
    for s in range(SPB):
        o_ref[s * SEG:(s + 1) * SEG, :] = (
            x_ref[s * SEG:(s + 1) * SEG, :]
            + _bf16_dot(ps[s], d[s * CAP:(s + 1) * CAP])
        )


def _dense_body(x_ref, m_ref, wt_ref, b_ref, g_ref, be_ref, o_ref):
    x = x_ref[...]
    soft = _bf16_dot(x.astype(jnp.bfloat16), wt_ref[...]) + b_ref[...]
    normed = _ln(x + soft, g_ref[...], be_ref[...])
    o_ref[...] = jnp.where(m_ref[...] == 1, normed, x)


def kernel(batch_embeddings, position_mask, W, b, gamma, beta):
    B, S, Hh = batch_embeddings.shape
    N = B * S
    x = batch_embeddings.reshape(N, Hh)
    mflat = position_mask.reshape(N).astype(jnp.int32)
    wt = W.T.astype(jnp.bfloat16)
    b2 = b.reshape(1, Hh)
    g2 = gamma.reshape(1, Hh)
    be2 = beta.reshape(1, Hh)

    sc = pl.kernel(
        _sc_compact,
        out_type=(
            jax.ShapeDtypeStruct((NW, 1, CAP), jnp.int32),
            jax.ShapeDtypeStruct((NW, L), jnp.int32),
        ),
        mesh=_SC_MESH,
        compiler_params=pltpu.CompilerParams(needs_layout_passes=False),
        scratch_types=[
            pltpu.VMEM((SEG,), jnp.int32),
            pltpu.VMEM((CAP,), jnp.int32),
            pltpu.VMEM((L,), jnp.int32),
        ],
    )
    idx, cnt = sc(mflat)

    def sparse_path(ops):
        xx, cc, ii = ops
        return pl.pallas_call(
            _sparse_body,
            grid=(N // BLK,),
            in_specs=[
                pl.BlockSpec(memory_space=pltpu.SMEM),
                pl.BlockSpec((BLK, Hh), lambda i: (i, 0)),
                pl.BlockSpec((SPB, 1, CAP), lambda i: (i, 0, 0)),
                pl.BlockSpec((Hh, Hh), lambda i: (0, 0)),
                pl.BlockSpec((1, Hh), lambda i: (0, 0)),
                pl.BlockSpec((1, Hh), lambda i: (0, 0)),
                pl.BlockSpec((1, Hh), lambda i: (0, 0)),
            ],
            out_specs=pl.BlockSpec((BLK, Hh), lambda i: (i, 0)),
            out_shape=jax.ShapeDtypeStruct((N, Hh), jnp.float32),
            compiler_params=pltpu.CompilerParams(
                dimension_semantics=("parallel",),
            ),
        )(cc, xx, ii, wt, b2, g2, be2)

    def dense_path(ops):
        xx, cc, ii = ops
        return pl.pallas_call(
            _dense_body,
            grid=(N // BLK,),
            in_specs=[
                pl.BlockSpec((BLK, Hh), lambda i: (i, 0)),
                pl.BlockSpec((BLK, 1), lambda i: (i, 0)),
                pl.BlockSpec((Hh, Hh), lambda i: (0, 0)),
                pl.BlockSpec((1, Hh), lambda i: (0, 0)),
                pl.BlockSpec((1, Hh), lambda i: (0, 0)),
                pl.BlockSpec((1, Hh), lambda i: (0, 0)),
            ],
            out_specs=pl.BlockSpec((BLK, Hh), lambda i: (i, 0)),
            out_shape=jax.ShapeDtypeStruct((N, Hh), jnp.float32),
            compiler_params=pltpu.CompilerParams(
                dimension_semantics=("arbitrary",),
            ),
        )(xx, mflat.reshape(N, 1), wt, b2, g2, be2)

    overflow = jnp.any(cnt[:, 0] > CAP)
    out = lax.cond(overflow, dense_path, sparse_path, (x, cnt, idx))
    return out.reshape(B, S, Hh)
